# baseline (speedup 1.0000x reference)
import functools

import jax
import jax.numpy as jnp
from jax import lax
from jax.experimental import pallas as pl
from jax.experimental.pallas import tpu as pltpu
from jax.experimental.pallas import tpu_sc as plsc

_VOCAB = 50000
_D = 768
_B = 1024
_S = 200
_LANES = 16
_NC = 2
_NS = 16
_NW = _NC * _NS
_BPW = _B // _NW
_NCHUNK = _D // _LANES
_SPD = 2
_G = _BPW * _SPD
_NDMA = _S // _SPD
_IDS_PW = _S * _BPW


def _pool_body(ids_hbm, emb_hbm, out_hbm, idx_v, rows0, rows1, acc_v, sem0, sem1):
    wid = lax.axis_index("s") * _NC + lax.axis_index("c")
    base = wid * _BPW

    pltpu.sync_copy(ids_hbm.at[pl.ds(wid * _IDS_PW, _IDS_PW)], idx_v)

    def _zero_row(i, _):
        for d in range(_NCHUNK):
            acc_v[i, pl.ds(d * _LANES, _LANES)] = jnp.zeros(
                (_LANES,), jnp.float32
            )
        return 0

    lax.fori_loop(0, _BPW, _zero_row, 0)

    def _start(k, buf, sem):
        pltpu.async_copy(emb_hbm.at[idx_v.at[pl.ds(k * _G, _G)]], buf, sem)

    def _drain(buf, sem):
        pltpu.make_async_copy(
            emb_hbm.at[idx_v.at[pl.ds(0, _G)]], buf, sem
        ).wait()

    def _accum(buf):
        def _acc_row(i, _):
            for d in range(_NCHUNK):
                sl = pl.ds(d * _LANES, _LANES)
                plsc.addupdate(
                    acc_v.at[i, sl], buf[i, sl] + buf[i + _BPW, sl]
                )
            return 0

        lax.fori_loop(0, _BPW, _acc_row, 0)

    _start(0, rows0, sem0)

    def _step(k, _):
        nxt = k + 1

        @pl.when(nxt < _NDMA)
        def _():
            @pl.when(nxt % 2 == 0)
            def _():
                _start(nxt, rows0, sem0)

            @pl.when(nxt % 2 == 1)
            def _():
                _start(nxt, rows1, sem1)

        @pl.when(k % 2 == 0)
        def _():
            _drain(rows0, sem0)
            _accum(rows0)

        @pl.when(k % 2 == 1)
        def _():
            _drain(rows1, sem1)
            _accum(rows1)

        return 0

    lax.fori_loop(0, _NDMA, _step, 0)

    pltpu.sync_copy(acc_v, out_hbm.at[pl.ds(base, _BPW)])


@jax.jit
def _pool(ids_flat, emb_table):
    return pl.kernel(
        _pool_body,
        out_type=jax.ShapeDtypeStruct((_B, _D), jnp.float32),
        mesh=plsc.VectorSubcoreMesh(core_axis_name="c", subcore_axis_name="s"),
        scratch_types=[
            pltpu.VMEM((_IDS_PW,), jnp.int32),
            pltpu.VMEM((_G, _D), jnp.float32),
            pltpu.VMEM((_G, _D), jnp.float32),
            pltpu.VMEM((_BPW, _D), jnp.float32),
            pltpu.SemaphoreType.DMA,
            pltpu.SemaphoreType.DMA,
        ],
    )(ids_flat, emb_table)


_BN = 2048
_NT = (_VOCAB + _BN - 1) // _BN


def _mm_body(x_ref, w_ref, b_ref, o_ref):
    acc = lax.dot_general(
        x_ref[...].astype(jnp.bfloat16),
        w_ref[...].astype(jnp.bfloat16),
        (((1,), (1,)), ((), ())),
        preferred_element_type=jnp.float32,
    )
    o_ref[...] = acc * (1.0 / _S) + b_ref[...]


@jax.jit
def _project(pooled, w, bias):
    return pl.pallas_call(
        _mm_body,
        grid=(_NT,),
        in_specs=[
            pl.BlockSpec((_B, _D), lambda j: (0, 0)),
            pl.BlockSpec((_BN, _D), lambda j: (j, 0)),
            pl.BlockSpec((1, _BN), lambda j: (0, j)),
        ],
        out_specs=pl.BlockSpec((_B, _BN), lambda j: (0, j)),
        out_shape=jax.ShapeDtypeStruct((_B, _VOCAB), jnp.float32),
    )(pooled, w, bias)


def kernel(input_ids, emb_table, W, b):
    ids_flat = (
        input_ids.T.reshape(_S, _NW, _BPW).transpose(1, 0, 2).reshape(-1)
    )
    sums = _pool(ids_flat, emb_table)
    return _project(sums, W, b.reshape(1, _VOCAB))


# speedup vs baseline: 1.1360x; 1.1360x over previous
import functools

import jax
import jax.numpy as jnp
from jax import lax
from jax.experimental import pallas as pl
from jax.experimental.pallas import tpu as pltpu
from jax.experimental.pallas import tpu_sc as plsc

_VOCAB = 50000
_D = 768
_B = 1024
_S = 200
_LANES = 16
_NC = 2
_NS = 16
_NW = _NC * _NS
_BPW = _B // _NW
_NCHUNK = _D // _LANES


def _pool_body(ids_hbm, emb_hbm, out_hbm, idx_v, rows0, rows1, acc_v, sem0, sem1):
    wid = lax.axis_index("s") * _NC + lax.axis_index("c")
    base = wid * _BPW

    pltpu.sync_copy(ids_hbm.at[pl.ds(wid * _S, _S)], idx_v)

    def _zero_row(i, _):
        for d in range(_NCHUNK):
            acc_v[i, pl.ds(d * _LANES, _LANES)] = jnp.zeros(
                (_LANES,), jnp.float32
            )
        return 0

    lax.fori_loop(0, _BPW, _zero_row, 0)

    def _start(s, buf, sem):
        pltpu.async_copy(emb_hbm.at[idx_v.at[s, pl.ds(0, _BPW)]], buf, sem)

    def _drain(buf, sem):
        pltpu.make_async_copy(
            emb_hbm.at[idx_v.at[0, pl.ds(0, _BPW)]], buf, sem
        ).wait()

    def _accum(buf):
        def _acc_row(i, _):
            for d in range(_NCHUNK):
                sl = pl.ds(d * _LANES, _LANES)
                plsc.addupdate(acc_v.at[i, sl], buf[i, sl])
            return 0

        lax.fori_loop(0, _BPW, _acc_row, 0)

    _start(0, rows0, sem0)

    def _step(s, _):
        nxt = s + 1

        @pl.when(nxt < _S)
        def _():
            @pl.when(nxt % 2 == 0)
            def _():
                _start(nxt, rows0, sem0)

            @pl.when(nxt % 2 == 1)
            def _():
                _start(nxt, rows1, sem1)

        @pl.when(s % 2 == 0)
        def _():
            _drain(rows0, sem0)
            _accum(rows0)

        @pl.when(s % 2 == 1)
        def _():
            _drain(rows1, sem1)
            _accum(rows1)

        return 0

    lax.fori_loop(0, _S, _step, 0)

    pltpu.sync_copy(acc_v, out_hbm.at[pl.ds(base, _BPW)])


@jax.jit
def _pool(ids_pad, emb_table):
    return pl.kernel(
        _pool_body,
        out_type=jax.ShapeDtypeStruct((_B, _D), jnp.float32),
        mesh=plsc.VectorSubcoreMesh(core_axis_name="c", subcore_axis_name="s"),
        scratch_types=[
            pltpu.VMEM((_S, 128), jnp.int32),
            pltpu.VMEM((_BPW, _D), jnp.float32),
            pltpu.VMEM((_BPW, _D), jnp.float32),
            pltpu.VMEM((_BPW, _D), jnp.float32),
            pltpu.SemaphoreType.DMA,
            pltpu.SemaphoreType.DMA,
        ],
    )(ids_pad, emb_table)


_BN = 2048
_NT = (_VOCAB + _BN - 1) // _BN


def _mm_body(x_ref, w_ref, b_ref, o_ref):
    acc = lax.dot_general(
        x_ref[...].astype(jnp.bfloat16),
        w_ref[...].astype(jnp.bfloat16),
        (((1,), (1,)), ((), ())),
        preferred_element_type=jnp.float32,
    )
    o_ref[...] = acc * (1.0 / _S) + b_ref[...]


@jax.jit
def _project(pooled, w, bias):
    return pl.pallas_call(
        _mm_body,
        grid=(_NT,),
        in_specs=[
            pl.BlockSpec((_B, _D), lambda j: (0, 0)),
            pl.BlockSpec((_BN, _D), lambda j: (j, 0)),
            pl.BlockSpec((1, _BN), lambda j: (0, j)),
        ],
        out_specs=pl.BlockSpec((_B, _BN), lambda j: (0, j)),
        out_shape=jax.ShapeDtypeStruct((_B, _VOCAB), jnp.float32),
    )(pooled, w, bias)


def kernel(input_ids, emb_table, W, b):
    ids_w = input_ids.T.reshape(_S, _NW, _BPW).transpose(1, 0, 2)
    ids_pad = jnp.pad(ids_w, ((0, 0), (0, 0), (0, 128 - _BPW)))
    ids_pad = ids_pad.reshape(_NW * _S, 128)
    sums = _pool(ids_pad, emb_table)
    return _project(sums, W, b.reshape(1, _VOCAB))


# speedup vs baseline: 1.1390x; 1.0026x over previous
import functools

import jax
import jax.numpy as jnp
from jax import lax
from jax.experimental import pallas as pl
from jax.experimental.pallas import tpu as pltpu
from jax.experimental.pallas import tpu_sc as plsc

_VOCAB = 50000
_D = 768
_B = 1024
_S = 200
_LANES = 16
_NC = 2
_NS = 16
_NW = _NC * _NS
_BPW = _B // _NW
_NCHUNK = _D // _LANES


def _pool_body(ids_hbm, emb_hbm, out_hbm, idx_v, rows0, rows1, acc_v, sem0, sem1):
    wid = lax.axis_index("s") * _NC + lax.axis_index("c")
    base = wid * _BPW

    pltpu.sync_copy(ids_hbm.at[pl.ds(wid * _S, _S)], idx_v)

    def _zero_row(i, _):
        for d in range(_NCHUNK):
            acc_v[i, pl.ds(d * _LANES, _LANES)] = jnp.zeros(
                (_LANES,), jnp.float32
            )
        return 0

    lax.fori_loop(0, _BPW, _zero_row, 0)

    def _start(s, buf, sem):
        pltpu.async_copy(emb_hbm.at[idx_v.at[s, pl.ds(0, _BPW)]], buf, sem)

    def _drain(buf, sem):
        pltpu.make_async_copy(
            emb_hbm.at[idx_v.at[0, pl.ds(0, _BPW)]], buf, sem
        ).wait()

    def _accum(buf):
        def _acc_row(i, _):
            for d in range(_NCHUNK):
                sl = pl.ds(d * _LANES, _LANES)
                plsc.addupdate(acc_v.at[i, sl], buf[i, sl])
            return 0

        lax.fori_loop(0, _BPW, _acc_row, 0)

    _start(0, rows0, sem0)

    def _step(s, _):
        nxt = s + 1

        @pl.when(nxt < _S)
        def _():
            @pl.when(nxt % 2 == 0)
            def _():
                _start(nxt, rows0, sem0)

            @pl.when(nxt % 2 == 1)
            def _():
                _start(nxt, rows1, sem1)

        @pl.when(s % 2 == 0)
        def _():
            _drain(rows0, sem0)
            _accum(rows0)

        @pl.when(s % 2 == 1)
        def _():
            _drain(rows1, sem1)
            _accum(rows1)

        return 0

    lax.fori_loop(0, _S, _step, 0)

    pltpu.sync_copy(acc_v, out_hbm.at[pl.ds(base, _BPW)])


@jax.jit
def _pool(ids_pad, emb_table):
    return pl.kernel(
        _pool_body,
        out_type=jax.ShapeDtypeStruct((_B, _D), jnp.float32),
        mesh=plsc.VectorSubcoreMesh(core_axis_name="c", subcore_axis_name="s"),
        scratch_types=[
            pltpu.VMEM((_S, 128), jnp.int32),
            pltpu.VMEM((_BPW, _D), jnp.float32),
            pltpu.VMEM((_BPW, _D), jnp.float32),
            pltpu.VMEM((_BPW, _D), jnp.float32),
            pltpu.SemaphoreType.DMA,
            pltpu.SemaphoreType.DMA,
        ],
    )(ids_pad, emb_table)


_BN = 3072
_NT = (_VOCAB + _BN - 1) // _BN


def _mm_body(x_ref, w_ref, b_ref, o_ref):
    acc = lax.dot_general(
        x_ref[...].astype(jnp.bfloat16),
        w_ref[...].astype(jnp.bfloat16),
        (((1,), (1,)), ((), ())),
        preferred_element_type=jnp.float32,
    )
    o_ref[...] = acc * (1.0 / _S) + b_ref[...]


@jax.jit
def _project(pooled, w, bias):
    return pl.pallas_call(
        _mm_body,
        grid=(_NT,),
        in_specs=[
            pl.BlockSpec((_B, _D), lambda j: (0, 0)),
            pl.BlockSpec((_BN, _D), lambda j: (j, 0)),
            pl.BlockSpec((1, _BN), lambda j: (0, j)),
        ],
        out_specs=pl.BlockSpec((_B, _BN), lambda j: (0, j)),
        out_shape=jax.ShapeDtypeStruct((_B, _VOCAB), jnp.float32),
    )(pooled, w, bias)


def kernel(input_ids, emb_table, W, b):
    ids_w = input_ids.T.reshape(_S, _NW, _BPW).transpose(1, 0, 2)
    ids_pad = jnp.pad(ids_w, ((0, 0), (0, 0), (0, 128 - _BPW)))
    ids_pad = ids_pad.reshape(_NW * _S, 128)
    sums = _pool(ids_pad, emb_table)
    return _project(sums, W, b.reshape(1, _VOCAB))


# speedup vs baseline: 1.1414x; 1.0021x over previous
import functools

import jax
import jax.numpy as jnp
from jax import lax
from jax.experimental import pallas as pl
from jax.experimental.pallas import tpu as pltpu
from jax.experimental.pallas import tpu_sc as plsc

_VOCAB = 50000
_D = 768
_B = 1024
_S = 200
_LANES = 16
_NC = 2
_NS = 16
_NW = _NC * _NS
_BPW = _B // _NW
_NCHUNK = _D // _LANES


def _pool_body(ids_hbm, emb_hbm, out_hbm, idx_v, rows0, rows1, acc_v, sem0, sem1):
    wid = lax.axis_index("s") * _NC + lax.axis_index("c")
    base = wid * _BPW

    pltpu.sync_copy(ids_hbm.at[pl.ds(wid * _S, _S)], idx_v)

    def _zero_row(i, _):
        for d in range(_NCHUNK):
            acc_v[i, pl.ds(d * _LANES, _LANES)] = jnp.zeros(
                (_LANES,), jnp.float32
            )
        return 0

    lax.fori_loop(0, _BPW, _zero_row, 0)

    def _start(s, buf, sem):
        pltpu.async_copy(emb_hbm.at[idx_v.at[s, pl.ds(0, _BPW)]], buf, sem)

    def _drain(buf, sem):
        pltpu.make_async_copy(
            emb_hbm.at[idx_v.at[0, pl.ds(0, _BPW)]], buf, sem
        ).wait()

    def _accum(buf):
        def _acc_row(i, _):
            for d in range(_NCHUNK):
                sl = pl.ds(d * _LANES, _LANES)
                plsc.addupdate(acc_v.at[i, sl], buf[i, sl])
            return 0

        lax.fori_loop(0, _BPW, _acc_row, 0)

    _start(0, rows0, sem0)

    def _step(s, _):
        nxt = s + 1

        @pl.when(nxt < _S)
        def _():
            @pl.when(nxt % 2 == 0)
            def _():
                _start(nxt, rows0, sem0)

            @pl.when(nxt % 2 == 1)
            def _():
                _start(nxt, rows1, sem1)

        @pl.when(s % 2 == 0)
        def _():
            _drain(rows0, sem0)
            _accum(rows0)

        @pl.when(s % 2 == 1)
        def _():
            _drain(rows1, sem1)
            _accum(rows1)

        return 0

    lax.fori_loop(0, _S, _step, 0)

    pltpu.sync_copy(acc_v, out_hbm.at[pl.ds(base, _BPW)])


@jax.jit
def _pool(ids_pad, emb_table):
    return pl.kernel(
        _pool_body,
        out_type=jax.ShapeDtypeStruct((_B, _D), jnp.float32),
        mesh=plsc.VectorSubcoreMesh(core_axis_name="c", subcore_axis_name="s"),
        scratch_types=[
            pltpu.VMEM((_S, 128), jnp.int32),
            pltpu.VMEM((_BPW, _D), jnp.float32),
            pltpu.VMEM((_BPW, _D), jnp.float32),
            pltpu.VMEM((_BPW, _D), jnp.float32),
            pltpu.SemaphoreType.DMA,
            pltpu.SemaphoreType.DMA,
        ],
    )(ids_pad, emb_table)


_BN = 3584
_NT = (_VOCAB + _BN - 1) // _BN


def _mm_body(x_ref, w_ref, b_ref, o_ref):
    acc = lax.dot_general(
        x_ref[...].astype(jnp.bfloat16),
        w_ref[...].astype(jnp.bfloat16),
        (((1,), (1,)), ((), ())),
        preferred_element_type=jnp.float32,
    )
    o_ref[...] = acc * (1.0 / _S) + b_ref[...]


@jax.jit
def _project(pooled, w, bias):
    return pl.pallas_call(
        _mm_body,
        grid=(_NT,),
        in_specs=[
            pl.BlockSpec((_B, _D), lambda j: (0, 0)),
            pl.BlockSpec((_BN, _D), lambda j: (j, 0)),
            pl.BlockSpec((1, _BN), lambda j: (0, j)),
        ],
        out_specs=pl.BlockSpec((_B, _BN), lambda j: (0, j)),
        out_shape=jax.ShapeDtypeStruct((_B, _VOCAB), jnp.float32),
    )(pooled, w, bias)


def kernel(input_ids, emb_table, W, b):
    ids_w = input_ids.T.reshape(_S, _NW, _BPW).transpose(1, 0, 2)
    ids_pad = jnp.pad(ids_w, ((0, 0), (0, 0), (0, 128 - _BPW)))
    ids_pad = ids_pad.reshape(_NW * _S, 128)
    sums = _pool(ids_pad, emb_table)
    return _project(sums, W, b.reshape(1, _VOCAB))
